# baseline (speedup 1.0000x reference)
import functools

import jax
import jax.numpy as jnp
from jax import lax
from jax.experimental import pallas as pl
from jax.experimental.pallas import tpu as pltpu
from jax.experimental.pallas import tpu_sc as plsc

_F = 26
_V = 100000
_D = 32
_B = 16384
_NW = 32
_BH = _B // 2

_mesh = plsc.VectorSubcoreMesh(core_axis_name="c", subcore_axis_name="s")


def _make_group_kernel(f0, nf):
    ncols = nf * _D

    @functools.partial(
        pl.kernel,
        mesh=_mesh,
        out_type=jax.ShapeDtypeStruct((ncols, _B), jnp.float32),
        compiler_params=pltpu.CompilerParams(
            use_tc_tiling_on_sc=False, needs_layout_passes=False),
        scratch_types=[
            pltpu.VMEM((_V,), jnp.float32),
            pltpu.VMEM((_BH,), jnp.int32),
            pltpu.VMEM((_BH,), jnp.float32),
            pltpu.SemaphoreType.DMA,
        ],
    )
    def group_kernel(xt_hbm, tab_hbm, out_hbm, slicev, xfv, colv, sem):
        wid = lax.axis_index("s") * 2 + lax.axis_index("c")

        def task_body(t, carry):
            c = t * _NW + wid
            f = c // _D
            d = c - f * _D
            pltpu.sync_copy(tab_hbm.at[f, d], slicev)

            def half_body(h, carry2):
                pltpu.sync_copy(xt_hbm.at[f0 + f, pl.ds(h * _BH, _BH)], xfv)

                def vec_body(k, carry3):
                    for u in range(8):
                        sl = pl.ds(k * 128 + u * 16, 16)
                        colv[sl] = plsc.load_gather(slicev, [xfv[sl]])
                    return carry3

                lax.fori_loop(0, _BH // 128, vec_body, 0)
                pltpu.sync_copy(colv, out_hbm.at[c, pl.ds(h * _BH, _BH)])
                return carry2

            lax.fori_loop(0, 2, half_body, 0)
            return carry

        lax.fori_loop(0, ncols // _NW, task_body, 0)

    return group_kernel


_GROUPS = ((0, 7), (7, 7), (14, 6), (20, 6))
_KERNELS = tuple(_make_group_kernel(f0, nf) for f0, nf in _GROUPS)


@jax.jit
def kernel(x, tables):
    xt = jnp.transpose(x)
    tt = jnp.transpose(tables, (0, 2, 1))
    outs = [
        k(xt, tt[f0:f0 + nf]) for k, (f0, nf) in zip(_KERNELS, _GROUPS)
    ]
    out_t = jnp.concatenate(outs, axis=0)
    return jnp.transpose(out_t)
